# Initial kernel scaffold; baseline (speedup 1.0000x reference)
import jax
import jax.numpy as jnp
from jax.experimental import pallas as pl


def kernel(u_values, v_values):
    raise NotImplementedError("write your pallas kernel here")


# speedup vs baseline: 230.8663x; 230.8663x over previous
import jax
import jax.numpy as jnp
from jax import lax
from jax.experimental import pallas as pl
from jax.experimental.pallas import tpu as pltpu

R = 4096
C = 128
N = R * C
LOGN = 19


def _sort_body(u_in, v_in, out_ref, uref, vref, nref):
    row = lax.broadcasted_iota(jnp.int32, (R, C), 0)
    col = lax.broadcasted_iota(jnp.int32, (R, C), 1)
    uref[...] = u_in[...]
    vref[...] = v_in[...]
    nref[...] = col * R + row

    def cmpex(X, P, tm):
        q = (X <= P) == tm
        return jnp.where(q, X, P)

    def lane_step(t, kb):
        j = kb - 1 - t
        m = jnp.int32(1) << (j - 12)
        n = nref[...]
        bj = (n >> j) & 1
        tm = ((bj ^ (n >> kb)) & 1) == 0
        low = bj == 0
        U = uref[...]
        V = vref[...]
        PU = jnp.where(low, pltpu.roll(U, C - m, 1), pltpu.roll(U, m, 1))
        PV = jnp.where(low, pltpu.roll(V, C - m, 1), pltpu.roll(V, m, 1))
        uref[...] = cmpex(U, PU, tm)
        vref[...] = cmpex(V, PV, tm)
        return kb

    def sub_step(t, kb):
        j = jnp.minimum(kb, 12) - 1 - t
        s = jnp.int32(1) << j
        n = nref[...]
        bj = (n >> j) & 1
        tm = ((bj ^ (n >> kb)) & 1) == 0
        low = bj == 0
        U = uref[...]
        V = vref[...]
        PU = jnp.where(low, pltpu.roll(U, R - s, 0), pltpu.roll(U, s, 0))
        PV = jnp.where(low, pltpu.roll(V, R - s, 0), pltpu.roll(V, s, 0))
        uref[...] = cmpex(U, PU, tm)
        vref[...] = cmpex(V, PV, tm)
        return kb

    def phase(kb, carry):
        kb = jnp.int32(kb)
        lax.fori_loop(0, jnp.maximum(kb - 12, 0), lane_step, kb)
        lax.fori_loop(0, jnp.minimum(kb, 12), sub_step, kb)
        return carry

    lax.fori_loop(1, LOGN + 1, phase, None)
    out_ref[0, 0] = jnp.sum(jnp.abs(uref[...] - vref[...]))


def kernel(u_values, v_values):
    u2 = u_values.reshape(R, C)
    v2 = v_values.reshape(R, C)
    out = pl.pallas_call(
        _sort_body,
        out_shape=jax.ShapeDtypeStruct((1, 1), jnp.float32),
        in_specs=[
            pl.BlockSpec(memory_space=pltpu.VMEM),
            pl.BlockSpec(memory_space=pltpu.VMEM),
        ],
        out_specs=pl.BlockSpec(memory_space=pltpu.SMEM),
        scratch_shapes=[
            pltpu.VMEM((R, C), jnp.float32),
            pltpu.VMEM((R, C), jnp.float32),
            pltpu.VMEM((R, C), jnp.int32),
        ],
    )(u2, v2)
    return out.reshape(())


# speedup vs baseline: 385.8342x; 1.6712x over previous
import jax
import jax.numpy as jnp
from jax import lax
from jax.experimental import pallas as pl
from jax.experimental.pallas import tpu as pltpu

R = 4096
C = 128
N = R * C
LOGN = 19
H = 512
G = R // H
LH = 9


def _reg_step(Uc, Vc, nc, j, kb, axis, size, shift_base):
    sh = jnp.int32(1) << (j - shift_base)
    bj = (nc >> j) & 1
    tm = ((bj ^ (nc >> kb)) & 1) == 0
    low = bj == 0

    def partner(X):
        return jnp.where(low, pltpu.roll(X, size - sh, axis), pltpu.roll(X, sh, axis))

    PU = partner(Uc)
    PV = partner(Vc)
    Uc = jnp.where((Uc <= PU) == tm, Uc, PU)
    Vc = jnp.where((Vc <= PV) == tm, Vc, PV)
    return Uc, Vc


def _sort_body(u_in, v_in, out_ref, uref, vref):
    rowh = lax.broadcasted_iota(jnp.int32, (H, C), 0)
    colh = lax.broadcasted_iota(jnp.int32, (H, C), 1)

    def chunk_n(g):
        return colh * R + g * H + rowh

    def passA(g, carry):
        Uc = u_in[pl.ds(g * H, H), :]
        Vc = v_in[pl.ds(g * H, H), :]
        nc = chunk_n(g)

        def phase(kb, UV):
            def jstep(t, UV):
                return _reg_step(UV[0], UV[1], nc, kb - 1 - t, kb, 0, H, 0)

            return lax.fori_loop(0, kb, jstep, UV)

        Uc, Vc = lax.fori_loop(1, LH + 1, phase, (Uc, Vc))
        uref[pl.ds(g * H, H), :] = Uc
        vref[pl.ds(g * H, H), :] = Vc
        return carry

    lax.fori_loop(0, G, passA, None)

    def main_phase(kb, carry):
        @pl.when(kb > 12)
        def _():
            def chunk_body(g, c2):
                Uc = uref[pl.ds(g * H, H), :]
                Vc = vref[pl.ds(g * H, H), :]
                nc = chunk_n(g)

                def jstep(t, UV):
                    return _reg_step(UV[0], UV[1], nc, kb - 1 - t, kb, 1, C, 12)

                Uc, Vc = lax.fori_loop(0, kb - 12, jstep, (Uc, Vc))
                uref[pl.ds(g * H, H), :] = Uc
                vref[pl.ds(g * H, H), :] = Vc
                return c2

            lax.fori_loop(0, G, chunk_body, None)

        def cross_j(tj, c2):
            j = jnp.minimum(kb, 12) - 1 - tj
            d = jnp.int32(1) << (j - LH)

            def gbody(gg, c3):
                c0 = ((gg & ~(d - 1)) << 1) | (gg & (d - 1))
                b0 = c0 * H
                b1 = b0 + d * H
                X0u = uref[pl.ds(b0, H), :]
                X1u = uref[pl.ds(b1, H), :]
                X0v = vref[pl.ds(b0, H), :]
                X1v = vref[pl.ds(b1, H), :]
                n0 = colh * R + b0 + rowh
                asc = ((n0 >> kb) & 1) == 0
                lou = jnp.minimum(X0u, X1u)
                hiu = jnp.maximum(X0u, X1u)
                lov = jnp.minimum(X0v, X1v)
                hiv = jnp.maximum(X0v, X1v)
                uref[pl.ds(b0, H), :] = jnp.where(asc, lou, hiu)
                uref[pl.ds(b1, H), :] = jnp.where(asc, hiu, lou)
                vref[pl.ds(b0, H), :] = jnp.where(asc, lov, hiv)
                vref[pl.ds(b1, H), :] = jnp.where(asc, hiv, lov)
                return c3

            lax.fori_loop(0, G // 2, gbody, None)
            return c2

        lax.fori_loop(0, jnp.minimum(kb, 12) - LH, cross_j, None)

        def chunk_body2(g, c2):
            Uc = uref[pl.ds(g * H, H), :]
            Vc = vref[pl.ds(g * H, H), :]
            nc = chunk_n(g)

            def jstep(t, UV):
                return _reg_step(UV[0], UV[1], nc, LH - 1 - t, kb, 0, H, 0)

            Uc, Vc = lax.fori_loop(0, LH, jstep, (Uc, Vc))
            uref[pl.ds(g * H, H), :] = Uc
            vref[pl.ds(g * H, H), :] = Vc
            return c2

        lax.fori_loop(0, G, chunk_body2, None)
        return carry

    lax.fori_loop(LH + 1, LOGN + 1, main_phase, None)

    def red(g, acc):
        Uc = uref[pl.ds(g * H, H), :]
        Vc = vref[pl.ds(g * H, H), :]
        return acc + jnp.sum(jnp.abs(Uc - Vc))

    out_ref[0, 0] = lax.fori_loop(0, G, red, jnp.float32(0.0))


def kernel(u_values, v_values):
    u2 = u_values.reshape(R, C)
    v2 = v_values.reshape(R, C)
    out = pl.pallas_call(
        _sort_body,
        out_shape=jax.ShapeDtypeStruct((1, 1), jnp.float32),
        in_specs=[
            pl.BlockSpec(memory_space=pltpu.VMEM),
            pl.BlockSpec(memory_space=pltpu.VMEM),
        ],
        out_specs=pl.BlockSpec(memory_space=pltpu.SMEM),
        scratch_shapes=[
            pltpu.VMEM((R, C), jnp.float32),
            pltpu.VMEM((R, C), jnp.float32),
        ],
    )(u2, v2)
    return out.reshape(())


# speedup vs baseline: 499.6278x; 1.2949x over previous
import jax
import jax.numpy as jnp
from jax import lax
from jax.experimental import pallas as pl
from jax.experimental.pallas import tpu as pltpu

R = 4096
C = 128
N = R * C
LOGN = 19
H = 128
G = R // H
LC = 7
LW = 14


def _lane_step(Uc, Vc, nc, j, kb):
    m = jnp.int32(1) << j
    bj = (nc >> j) & 1
    tm = ((bj ^ (nc >> kb)) & 1) == 0
    low = bj == 0

    def partner(X):
        return jnp.where(low, pltpu.roll(X, C - m, 1), pltpu.roll(X, m, 1))

    PU = partner(Uc)
    PV = partner(Vc)
    Uc = jnp.where((Uc <= PU) == tm, Uc, PU)
    Vc = jnp.where((Vc <= PV) == tm, Vc, PV)
    return Uc, Vc


def _row_step(Uc, Vc, nc, j, kb):
    s = jnp.int32(1) << (j - LC)
    bj = (nc >> j) & 1
    tm = ((bj ^ (nc >> kb)) & 1) == 0
    low = bj == 0

    def partner(X):
        return jnp.where(low, pltpu.roll(X, H - s, 0), pltpu.roll(X, s, 0))

    PU = partner(Uc)
    PV = partner(Vc)
    Uc = jnp.where((Uc <= PU) == tm, Uc, PU)
    Vc = jnp.where((Vc <= PV) == tm, Vc, PV)
    return Uc, Vc


def _phase_in_chunk(Uc, Vc, nc, kb):
    jr_hi = jnp.minimum(kb, LW)

    def rstep(t, UV):
        return _row_step(UV[0], UV[1], nc, jr_hi - 1 - t, kb)

    Uc, Vc = lax.fori_loop(0, jnp.maximum(jr_hi - LC, 0), rstep, (Uc, Vc))
    jl_hi = jnp.minimum(kb, LC)

    def lstep(t, UV):
        return _lane_step(UV[0], UV[1], nc, jl_hi - 1 - t, kb)

    return lax.fori_loop(0, jl_hi, lstep, (Uc, Vc))


def _sort_body(u_in, v_in, out_ref, uref, vref):
    rowh = lax.broadcasted_iota(jnp.int32, (H, C), 0)
    colh = lax.broadcasted_iota(jnp.int32, (H, C), 1)

    def chunk_n(g):
        return (g * H + rowh) * C + colh

    def passA(g, carry):
        Uc = u_in[pl.ds(g * H, H), :]
        Vc = v_in[pl.ds(g * H, H), :]
        nc = chunk_n(g)

        def phase(kb, UV):
            return _phase_in_chunk(UV[0], UV[1], nc, kb)

        Uc, Vc = lax.fori_loop(1, LW + 1, phase, (Uc, Vc))
        uref[pl.ds(g * H, H), :] = Uc
        vref[pl.ds(g * H, H), :] = Vc
        return carry

    lax.fori_loop(0, G, passA, None)

    def main_phase(kb, carry):
        def cross_j(tj, c2):
            j = kb - 1 - tj
            d = jnp.int32(1) << (j - LW)

            def gbody(gg, c3):
                c0 = ((gg & ~(d - 1)) << 1) | (gg & (d - 1))
                b0 = c0 * H
                b1 = b0 + d * H
                asc = ((c0 >> (kb - LW)) & 1) == 0
                X0u = uref[pl.ds(b0, H), :]
                X1u = uref[pl.ds(b1, H), :]
                lou = jnp.minimum(X0u, X1u)
                hiu = jnp.maximum(X0u, X1u)
                uref[pl.ds(b0, H), :] = jnp.where(asc, lou, hiu)
                uref[pl.ds(b1, H), :] = jnp.where(asc, hiu, lou)
                X0v = vref[pl.ds(b0, H), :]
                X1v = vref[pl.ds(b1, H), :]
                lov = jnp.minimum(X0v, X1v)
                hiv = jnp.maximum(X0v, X1v)
                vref[pl.ds(b0, H), :] = jnp.where(asc, lov, hiv)
                vref[pl.ds(b1, H), :] = jnp.where(asc, hiv, lov)
                return c3

            lax.fori_loop(0, G // 2, gbody, None)
            return c2

        lax.fori_loop(0, kb - LW, cross_j, None)

        def chunk_body(g, c2):
            Uc = uref[pl.ds(g * H, H), :]
            Vc = vref[pl.ds(g * H, H), :]
            Uc, Vc = _phase_in_chunk(Uc, Vc, chunk_n(g), kb)
            uref[pl.ds(g * H, H), :] = Uc
            vref[pl.ds(g * H, H), :] = Vc
            return c2

        lax.fori_loop(0, G, chunk_body, None)
        return carry

    lax.fori_loop(LW + 1, LOGN + 1, main_phase, None)

    def red(g, acc):
        Uc = uref[pl.ds(g * H, H), :]
        Vc = vref[pl.ds(g * H, H), :]
        return acc + jnp.sum(jnp.abs(Uc - Vc))

    out_ref[0, 0] = lax.fori_loop(0, G, red, jnp.float32(0.0))


def kernel(u_values, v_values):
    u2 = u_values.reshape(R, C)
    v2 = v_values.reshape(R, C)
    out = pl.pallas_call(
        _sort_body,
        out_shape=jax.ShapeDtypeStruct((1, 1), jnp.float32),
        in_specs=[
            pl.BlockSpec(memory_space=pltpu.VMEM),
            pl.BlockSpec(memory_space=pltpu.VMEM),
        ],
        out_specs=pl.BlockSpec(memory_space=pltpu.SMEM),
        scratch_shapes=[
            pltpu.VMEM((R, C), jnp.float32),
            pltpu.VMEM((R, C), jnp.float32),
        ],
    )(u2, v2)
    return out.reshape(())


# speedup vs baseline: 1007.4454x; 2.0164x over previous
import jax
import jax.numpy as jnp
from jax import lax
from jax.experimental import pallas as pl
from jax.experimental.pallas import tpu as pltpu

R = 4096
C = 128
N = R * C
LOGN = 19
H = 128
G = R // H
LC = 7
LW = 14


def _step(Uc, Vc, nc, j, kb):
    bj = (nc >> j) & 1
    tm = ((bj ^ (nc >> kb)) & 1) == 0
    low = bj == 0
    if j >= LC:
        s = 1 << (j - LC)
        rolls = (H - s, s)
        axis = 0
    else:
        s = 1 << j
        rolls = (C - s, s)
        axis = 1

    def partner(X):
        return jnp.where(low, pltpu.roll(X, rolls[0], axis), pltpu.roll(X, rolls[1], axis))

    PU = partner(Uc)
    PV = partner(Vc)
    Uc = jnp.where((Uc <= PU) == tm, Uc, PU)
    Vc = jnp.where((Vc <= PV) == tm, Vc, PV)
    return Uc, Vc


def _phase_in_chunk(Uc, Vc, nc, kb):
    for j in range(min(kb, LW) - 1, -1, -1):
        Uc, Vc = _step(Uc, Vc, nc, j, kb)
    return Uc, Vc


def _sort_body(u_in, v_in, out_ref, uref, vref):
    rowh = lax.broadcasted_iota(jnp.int32, (H, C), 0)
    colh = lax.broadcasted_iota(jnp.int32, (H, C), 1)

    def chunk_n(g):
        return (g * H + rowh) * C + colh

    def passA(g, carry):
        Uc = u_in[pl.ds(g * H, H), :]
        Vc = v_in[pl.ds(g * H, H), :]
        nc = chunk_n(g)
        for kb in range(1, LW + 1):
            Uc, Vc = _phase_in_chunk(Uc, Vc, nc, kb)
        uref[pl.ds(g * H, H), :] = Uc
        vref[pl.ds(g * H, H), :] = Vc
        return carry

    lax.fori_loop(0, G, passA, None)

    for kb in range(LW + 1, LOGN + 1):
        for j in range(kb - 1, LW - 1, -1):
            d = 1 << (j - LW)

            def gbody(gg, c3, _d=d, _kb=kb):
                c0 = ((gg & ~(_d - 1)) << 1) | (gg & (_d - 1))
                b0 = c0 * H
                b1 = b0 + _d * H
                asc = ((c0 >> (_kb - LW)) & 1) == 0
                X0u = uref[pl.ds(b0, H), :]
                X1u = uref[pl.ds(b1, H), :]
                lou = jnp.minimum(X0u, X1u)
                hiu = jnp.maximum(X0u, X1u)
                uref[pl.ds(b0, H), :] = jnp.where(asc, lou, hiu)
                uref[pl.ds(b1, H), :] = jnp.where(asc, hiu, lou)
                X0v = vref[pl.ds(b0, H), :]
                X1v = vref[pl.ds(b1, H), :]
                lov = jnp.minimum(X0v, X1v)
                hiv = jnp.maximum(X0v, X1v)
                vref[pl.ds(b0, H), :] = jnp.where(asc, lov, hiv)
                vref[pl.ds(b1, H), :] = jnp.where(asc, hiv, lov)
                return c3

            lax.fori_loop(0, G // 2, gbody, None)

        def chunk_body(g, c2, _kb=kb):
            Uc = uref[pl.ds(g * H, H), :]
            Vc = vref[pl.ds(g * H, H), :]
            Uc, Vc = _phase_in_chunk(Uc, Vc, chunk_n(g), _kb)
            uref[pl.ds(g * H, H), :] = Uc
            vref[pl.ds(g * H, H), :] = Vc
            return c2

        lax.fori_loop(0, G, chunk_body, None)

    def red(g, acc):
        Uc = uref[pl.ds(g * H, H), :]
        Vc = vref[pl.ds(g * H, H), :]
        return acc + jnp.sum(jnp.abs(Uc - Vc))

    out_ref[0, 0] = lax.fori_loop(0, G, red, jnp.float32(0.0))


def kernel(u_values, v_values):
    u2 = u_values.reshape(R, C)
    v2 = v_values.reshape(R, C)
    out = pl.pallas_call(
        _sort_body,
        out_shape=jax.ShapeDtypeStruct((1, 1), jnp.float32),
        in_specs=[
            pl.BlockSpec(memory_space=pltpu.VMEM),
            pl.BlockSpec(memory_space=pltpu.VMEM),
        ],
        out_specs=pl.BlockSpec(memory_space=pltpu.SMEM),
        scratch_shapes=[
            pltpu.VMEM((R, C), jnp.float32),
            pltpu.VMEM((R, C), jnp.float32),
        ],
    )(u2, v2)
    return out.reshape(())


# speedup vs baseline: 1009.8411x; 1.0024x over previous
import jax
import jax.numpy as jnp
from jax import lax
from jax.experimental import pallas as pl
from jax.experimental.pallas import tpu as pltpu

R = 4096
C = 128
N = R * C
LOGN = 19
H = 128
G = R // H
LC = 7
LW = 14


def _step(Uc, Vc, nc, j, kb):
    bj = (nc >> j) & 1
    tm = ((bj ^ (nc >> kb)) & 1) == 0
    low = bj == 0
    if j >= LC:
        s = 1 << (j - LC)
        rolls = (H - s, s)
        axis = 0
    else:
        s = 1 << j
        rolls = (C - s, s)
        axis = 1

    def partner(X):
        return jnp.where(low, pltpu.roll(X, rolls[0], axis), pltpu.roll(X, rolls[1], axis))

    PU = partner(Uc)
    PV = partner(Vc)
    Uc = jnp.where((Uc <= PU) == tm, Uc, PU)
    Vc = jnp.where((Vc <= PV) == tm, Vc, PV)
    return Uc, Vc


def _phase_in_chunk(Uc, Vc, nc, kb):
    for j in range(min(kb, LW) - 1, -1, -1):
        Uc, Vc = _step(Uc, Vc, nc, j, kb)
    return Uc, Vc


def _sort_body(u_in, v_in, out_ref, uref, vref):
    rowh = lax.broadcasted_iota(jnp.int32, (H, C), 0)
    colh = lax.broadcasted_iota(jnp.int32, (H, C), 1)

    def chunk_n(g):
        return (g * H + rowh) * C + colh

    def passA(g, carry):
        Uc = u_in[pl.ds(g * H, H), :]
        Vc = v_in[pl.ds(g * H, H), :]
        nc = chunk_n(g)
        for kb in range(1, LW + 1):
            Uc, Vc = _phase_in_chunk(Uc, Vc, nc, kb)
        uref[pl.ds(g * H, H), :] = Uc
        vref[pl.ds(g * H, H), :] = Vc
        return carry

    lax.fori_loop(0, G, passA, None)

    for kb in range(LW + 1, LOGN + 1):
        for j in range(kb - 1, LW - 1, -1):
            d = 1 << (j - LW)

            def gbody(gg, c3, _d=d, _kb=kb):
                c0 = ((gg & ~(_d - 1)) << 1) | (gg & (_d - 1))
                b0 = c0 * H
                b1 = b0 + _d * H
                asc = ((c0 >> (_kb - LW)) & 1) == 0
                X0u = uref[pl.ds(b0, H), :]
                X1u = uref[pl.ds(b1, H), :]
                lou = jnp.minimum(X0u, X1u)
                hiu = jnp.maximum(X0u, X1u)
                uref[pl.ds(b0, H), :] = jnp.where(asc, lou, hiu)
                uref[pl.ds(b1, H), :] = jnp.where(asc, hiu, lou)
                X0v = vref[pl.ds(b0, H), :]
                X1v = vref[pl.ds(b1, H), :]
                lov = jnp.minimum(X0v, X1v)
                hiv = jnp.maximum(X0v, X1v)
                vref[pl.ds(b0, H), :] = jnp.where(asc, lov, hiv)
                vref[pl.ds(b1, H), :] = jnp.where(asc, hiv, lov)
                return c3

            lax.fori_loop(0, G // 2, gbody, None)

        last = kb == LOGN

        def chunk_body(g, acc, _kb=kb, _last=last):
            Uc = uref[pl.ds(g * H, H), :]
            Vc = vref[pl.ds(g * H, H), :]
            Uc, Vc = _phase_in_chunk(Uc, Vc, chunk_n(g), _kb)
            if _last:
                return acc + jnp.sum(jnp.abs(Uc - Vc))
            uref[pl.ds(g * H, H), :] = Uc
            vref[pl.ds(g * H, H), :] = Vc
            return acc

        total = lax.fori_loop(0, G, chunk_body, jnp.float32(0.0))

    out_ref[0, 0] = total


def kernel(u_values, v_values):
    u2 = u_values.reshape(R, C)
    v2 = v_values.reshape(R, C)
    out = pl.pallas_call(
        _sort_body,
        out_shape=jax.ShapeDtypeStruct((1, 1), jnp.float32),
        in_specs=[
            pl.BlockSpec(memory_space=pltpu.VMEM),
            pl.BlockSpec(memory_space=pltpu.VMEM),
        ],
        out_specs=pl.BlockSpec(memory_space=pltpu.SMEM),
        scratch_shapes=[
            pltpu.VMEM((R, C), jnp.float32),
            pltpu.VMEM((R, C), jnp.float32),
        ],
    )(u2, v2)
    return out.reshape(())
